# baseline (speedup 1.0000x reference)
import dataclasses
import functools

import jax
import jax.numpy as jnp
from jax import lax
from jax.experimental import pallas as pl
from jax.experimental.pallas import tpu as pltpu
from jax.experimental.pallas import tpu_sc as plsc

N = 10000
E = 320000
EL = 200000
D = 128

NC = 2
NS = 16
NW = NC * NS

NPAD = 10240
RPT = NPAD // NS

EPW = E // NW
ECH = 128
EMAIN = (EPW // ECH) // 2 * 2
ETAIL = EPW - EMAIN * ECH

LCH = 64
LNCH = EL // LCH
LEXTRA = LNCH % NW
LBASE = LNCH // NW

_f32 = jnp.float32


def _mesh():
    return plsc.VectorSubcoreMesh(core_axis_name="c", subcore_axis_name="s")


def _sc_params():
    cp = pltpu.CompilerParams()
    if "needs_layout_passes" in pltpu.CompilerParams.__dataclass_fields__:
        cp = dataclasses.replace(cp, needs_layout_passes=False)
    return cp


def _make_agg(want_deg):
    out_types = [jax.ShapeDtypeStruct((NC, NPAD, D), _f32)]
    if want_deg:
        out_types.append(jax.ShapeDtypeStruct((NW, NPAD), _f32))
    scratch_types = [
        pltpu.VMEM((2, ECH), jnp.int32),
        pltpu.VMEM((2, ECH), jnp.int32),
        pltpu.VMEM((1, ETAIL), jnp.int32),
        pltpu.VMEM((1, ETAIL), jnp.int32),
        pltpu.VMEM((2, ECH, D), _f32),
        pltpu.VMEM((ETAIL, D), _f32),
        pltpu.VMEM((16, D), _f32),
        pltpu.VMEM_SHARED((NPAD, D), _f32),
        pltpu.SemaphoreType.DMA,
        pltpu.SemaphoreType.DMA,
        pltpu.SemaphoreType.DMA,
        pltpu.SemaphoreType.DMA,
        pltpu.SemaphoreType.DMA,
    ]
    if want_deg:
        scratch_types.append(pltpu.VMEM((NPAD,), _f32))

    def body(src_hbm, dst_hbm, table_hbm, *rest):
        if want_deg:
            (acc_out, deg_out, sidx, didx, tsidx, tdidx, rows, trows, zblk,
             acc_sh, gsem0, gsem1, ssem0, ssem1, zsem, degloc) = rest
        else:
            (acc_out, sidx, didx, tsidx, tdidx, rows, trows, zblk,
             acc_sh, gsem0, gsem1, ssem0, ssem1, zsem) = rest
        c = lax.axis_index("c")
        s = lax.axis_index("s")
        wid = s * NC + c
        ebase = wid * EPW
        tb = s * RPT
        gsems = (gsem0, gsem1)
        ssems = (ssem0, ssem1)
        ones16 = jnp.ones((16,), _f32)

        @pl.loop(0, 16)
        def _(r):
            @pl.loop(0, D, step=16)
            def _(j):
                zblk[r, pl.ds(j, 16)] = jnp.zeros((16,), _f32)

        @pl.loop(0, RPT // 16)
        def _(r):
            pltpu.make_async_copy(
                zblk, acc_sh.at[pl.ds(tb + r * 16, 16)], zsem).start()

        if want_deg:
            @pl.loop(0, NPAD, step=16)
            def _(i):
                degloc[pl.ds(i, 16)] = jnp.zeros((16,), _f32)

        @pl.loop(0, RPT // 16)
        def _(r):
            pltpu.make_async_copy(
                zblk, acc_sh.at[pl.ds(tb + r * 16, 16)], zsem).wait()

        plsc.subcore_barrier()

        def load_idx(b, j):
            off = ebase + j * ECH
            pltpu.sync_copy(src_hbm.at[pl.ds(off, ECH)], sidx.at[b])
            pltpu.sync_copy(dst_hbm.at[pl.ds(off, ECH)], didx.at[b])

        def g_desc(b):
            return pltpu.make_async_copy(
                table_hbm.at[sidx.at[b]], rows.at[b], gsems[b])

        def s_desc(b):
            return pltpu.make_async_copy(
                rows.at[b], acc_sh.at[didx.at[b]], ssems[b])

        def deg_upd(b):
            if want_deg:
                for j in range(ECH // 16):
                    dv = didx[b, pl.ds(j * 16, 16)]
                    plsc.addupdate_scatter(degloc, [dv], ones16)

        def pair_body(p, first, prefetch):
            g_desc(0).wait()
            s_desc(0).start(add=True)
            deg_upd(0)
            if not first:
                s_desc(1).wait()
            load_idx(1, 2 * p + 1)
            g_desc(1).start()
            s_desc(0).wait()
            if prefetch:
                load_idx(0, 2 * p + 2)
                g_desc(0).start()
            g_desc(1).wait()
            s_desc(1).start(add=True)
            deg_upd(1)

        load_idx(0, 0)
        g_desc(0).start()
        pair_body(0, first=True, prefetch=True)

        @pl.loop(1, EMAIN // 2 - 1)
        def _(p):
            pair_body(p, first=False, prefetch=True)

        pair_body(EMAIN // 2 - 1, first=False, prefetch=False)
        s_desc(1).wait()

        toff = ebase + EMAIN * ECH
        pltpu.sync_copy(src_hbm.at[pl.ds(toff, ETAIL)], tsidx.at[0])
        pltpu.sync_copy(dst_hbm.at[pl.ds(toff, ETAIL)], tdidx.at[0])
        pltpu.sync_copy(table_hbm.at[tsidx.at[0]], trows)
        pltpu.sync_copy(trows, acc_sh.at[tdidx.at[0]], add=True)
        if want_deg:
            plsc.addupdate_scatter(degloc, [tdidx[0]], ones16[:ETAIL])

        plsc.subcore_barrier()

        pltpu.sync_copy(acc_sh.at[pl.ds(tb, RPT)],
                        acc_out.at[c, pl.ds(tb, RPT)])
        if want_deg:
            pltpu.sync_copy(degloc, deg_out.at[wid])

    return pl.kernel(body, out_type=out_types, mesh=_mesh(),
                     scratch_types=scratch_types,
                     compiler_params=_sc_params())


_agg_with_deg = _make_agg(True)
_agg_plain = _make_agg(False)


def _dense_body(relu, residual, acc0, acc1, deg, x, wlt, wrt, b, out):
    d = jnp.maximum(jnp.sum(deg[...], axis=1), 1.0)
    agg = (acc0[...] + acc1[...]) / d[:, None]
    h = (jnp.dot(agg, wlt[...], preferred_element_type=_f32)
         + jnp.dot(x[...], wrt[...], preferred_element_type=_f32)
         + b[...])
    if residual:
        h = h + x[...]
    if relu:
        h = jnp.maximum(h, 0.0)
    out[...] = h.astype(out.dtype)


def _dense(acc_p, deg_p, x, w_l, w_r, b, relu, residual, out_dtype=_f32):
    blk = 1000
    grid = (N // blk,)
    body = functools.partial(_dense_body, relu, residual)
    return pl.pallas_call(
        body,
        grid=grid,
        in_specs=[
            pl.BlockSpec((blk, D), lambda i: (i, 0)),
            pl.BlockSpec((blk, D), lambda i: (i, 0)),
            pl.BlockSpec((blk, NW), lambda i: (i, 0)),
            pl.BlockSpec((blk, D), lambda i: (i, 0)),
            pl.BlockSpec((D, D), lambda i: (0, 0)),
            pl.BlockSpec((D, D), lambda i: (0, 0)),
            pl.BlockSpec((1, D), lambda i: (0, 0)),
        ],
        out_specs=pl.BlockSpec((blk, D), lambda i: (i, 0)),
        out_shape=jax.ShapeDtypeStruct((N, D), out_dtype),
    )(acc_p[0], acc_p[1], deg_p.T, x, w_l.T, w_r.T, b.reshape(1, D))


def _decode_body(a_hbm, b_hbm, table_hbm, out_hbm, aidx, bidx, arows, brows,
                 outv, table_sh, gsem0, gsem1, isem0, isem1,
                 osem0, osem1, tsem):
    c = lax.axis_index("c")
    s = lax.axis_index("s")
    wid = s * NC + c
    nch = LBASE + jnp.where(wid < LEXTRA, 1, 0)
    gsems = (gsem0, gsem1)
    isems = (isem0, isem1)
    osems = (osem0, osem1)
    lanes = jax.lax.iota(jnp.int32, 16)

    def goff(i):
        return (wid + i * NW) * LCH

    def ia_desc(b, i):
        return pltpu.make_async_copy(
            a_hbm.at[pl.ds(goff(i), LCH)], aidx.at[b], isems[b])

    def ib_desc(b, i):
        return pltpu.make_async_copy(
            b_hbm.at[pl.ds(goff(i), LCH)], bidx.at[b], isems[b])

    def ga_desc(b):
        return pltpu.make_async_copy(
            table_sh.at[aidx.at[b]], arows.at[b], gsems[b])

    def gb_desc(b):
        return pltpu.make_async_copy(
            table_sh.at[bidx.at[b]], brows.at[b], gsems[b])

    def o_desc(b, i):
        return pltpu.make_async_copy(
            outv.at[b], out_hbm.at[pl.ds(goff(i), LCH)], osems[b])

    def compute(b):
        @pl.loop(0, LCH, step=16)
        def _(r0):
            res = jnp.zeros((16,), _f32)
            for k in range(16):
                r = r0 + k
                acc = arows[b, r, pl.ds(0, 16)] * brows[b, r, pl.ds(0, 16)]
                for j in range(1, D // 16):
                    acc = acc + (arows[b, r, pl.ds(j * 16, 16)]
                                 * brows[b, r, pl.ds(j * 16, 16)])
                res = jnp.where(lanes == k, jnp.sum(acc), res)
            outv[b, pl.ds(r0, 16)] = res

    def slot(j, b):
        @pl.when(j < nch)
        def _():
            @pl.when(j >= 2)
            def _():
                o_desc(b, j - 2).wait()

            ga_desc(b).wait()
            gb_desc(b).wait()

            @pl.when(j + 2 < nch)
            def _():
                ia_desc(b, j + 2).start()
                ib_desc(b, j + 2).start()

            b1 = b ^ 1

            @pl.when(j + 1 < nch)
            def _():
                ia_desc(b1, j + 1).wait()
                ib_desc(b1, j + 1).wait()
                ga_desc(b1).start()
                gb_desc(b1).start()

            compute(b)
            o_desc(b, j).start()

    srows = N // NS // 8 * 8 + 8
    lrows = N - 15 * srows

    @pl.when(s < 15)
    def _():
        pltpu.make_async_copy(table_hbm.at[pl.ds(s * srows, srows)],
                              table_sh.at[pl.ds(s * srows, srows)],
                              tsem).start()

    @pl.when(s == 15)
    def _():
        pltpu.make_async_copy(table_hbm.at[pl.ds(15 * srows, lrows)],
                              table_sh.at[pl.ds(15 * srows, lrows)],
                              tsem).start()

    for b in range(2):
        ia_desc(b, b).start()
        ib_desc(b, b).start()

    @pl.when(s < 15)
    def _():
        pltpu.make_async_copy(table_hbm.at[pl.ds(s * srows, srows)],
                              table_sh.at[pl.ds(s * srows, srows)],
                              tsem).wait()

    @pl.when(s == 15)
    def _():
        pltpu.make_async_copy(table_hbm.at[pl.ds(15 * srows, lrows)],
                              table_sh.at[pl.ds(15 * srows, lrows)],
                              tsem).wait()

    plsc.subcore_barrier()

    ia_desc(0, 0).wait()
    ib_desc(0, 0).wait()
    ga_desc(0).start()
    gb_desc(0).start()

    @pl.loop(0, (LBASE + 1) // 2)
    def _(t):
        j = t * 2
        slot(j, 0)
        slot(j + 1, 1)

    o_desc(0, LBASE - 1).wait()

    @pl.when(nch > LBASE)
    def _():
        o_desc(1, LBASE).wait()

    @pl.when(nch <= LBASE)
    def _():
        o_desc(1, LBASE - 2).wait()


_decode = pl.kernel(
    _decode_body,
    out_type=jax.ShapeDtypeStruct((EL,), _f32),
    mesh=_mesh(),
    scratch_types=[
        pltpu.VMEM((2, LCH), jnp.int32),
        pltpu.VMEM((2, LCH), jnp.int32),
        pltpu.VMEM((2, LCH, D), _f32),
        pltpu.VMEM((2, LCH, D), _f32),
        pltpu.VMEM((2, LCH), _f32),
        pltpu.VMEM_SHARED((N, D), _f32),
        pltpu.SemaphoreType.DMA,
        pltpu.SemaphoreType.DMA,
        pltpu.SemaphoreType.DMA,
        pltpu.SemaphoreType.DMA,
        pltpu.SemaphoreType.DMA,
        pltpu.SemaphoreType.DMA,
        pltpu.SemaphoreType.DMA,
    ],
    compiler_params=_sc_params(),
)


def kernel(node_features, edge_index, edge_label_index,
           W1_l, b1_l, W1_r, W2_l, b2_l, W2_r):
    src = edge_index[0]
    dst = edge_index[1]
    acc1, deg = _agg_with_deg(src, dst, node_features)
    h1 = _dense(acc1, deg, node_features, W1_l, W1_r, b1_l,
                relu=True, residual=False)
    (acc2,) = _agg_plain(src, dst, h1)
    h2 = _dense(acc2, deg, h1, W2_l, W2_r, b2_l, relu=False, residual=True)
    return _decode(edge_label_index[0], edge_label_index[1], h2)


# speedup vs baseline: 1.0243x; 1.0243x over previous
import dataclasses
import functools

import jax
import jax.numpy as jnp
from jax import lax
from jax.experimental import pallas as pl
from jax.experimental.pallas import tpu as pltpu
from jax.experimental.pallas import tpu_sc as plsc

N = 10000
E = 320000
EL = 200000
D = 128

NC = 2
NS = 16
NW = NC * NS

NPAD = 10240
RPT = NPAD // NS

EPW = E // NW
ECH = 128
EMAIN = (EPW // ECH) // 2 * 2
ETAIL = EPW - EMAIN * ECH

LCH = 80
LNCH = EL // LCH
LEXTRA = LNCH % NW
LBASE = LNCH // NW

_f32 = jnp.float32


def _mesh():
    return plsc.VectorSubcoreMesh(core_axis_name="c", subcore_axis_name="s")


def _sc_params():
    cp = pltpu.CompilerParams()
    if "needs_layout_passes" in pltpu.CompilerParams.__dataclass_fields__:
        cp = dataclasses.replace(cp, needs_layout_passes=False)
    return cp


def _make_agg(want_deg):
    out_types = [jax.ShapeDtypeStruct((NC, NPAD, D), _f32)]
    if want_deg:
        out_types.append(jax.ShapeDtypeStruct((NW, NPAD), _f32))
    scratch_types = [
        pltpu.VMEM((2, ECH), jnp.int32),
        pltpu.VMEM((2, ECH), jnp.int32),
        pltpu.VMEM((1, ETAIL), jnp.int32),
        pltpu.VMEM((1, ETAIL), jnp.int32),
        pltpu.VMEM((2, ECH, D), _f32),
        pltpu.VMEM((ETAIL, D), _f32),
        pltpu.VMEM((16, D), _f32),
        pltpu.VMEM_SHARED((NPAD, D), _f32),
        pltpu.SemaphoreType.DMA,
        pltpu.SemaphoreType.DMA,
        pltpu.SemaphoreType.DMA,
        pltpu.SemaphoreType.DMA,
        pltpu.SemaphoreType.DMA,
    ]
    if want_deg:
        scratch_types.append(pltpu.VMEM((NPAD,), _f32))

    def body(src_hbm, dst_hbm, table_hbm, *rest):
        if want_deg:
            (acc_out, deg_out, sidx, didx, tsidx, tdidx, rows, trows, zblk,
             acc_sh, gsem0, gsem1, ssem0, ssem1, zsem, degloc) = rest
        else:
            (acc_out, sidx, didx, tsidx, tdidx, rows, trows, zblk,
             acc_sh, gsem0, gsem1, ssem0, ssem1, zsem) = rest
        c = lax.axis_index("c")
        s = lax.axis_index("s")
        wid = s * NC + c
        ebase = wid * EPW
        tb = s * RPT
        gsems = (gsem0, gsem1)
        ssems = (ssem0, ssem1)
        ones16 = jnp.ones((16,), _f32)

        @pl.loop(0, 16)
        def _(r):
            @pl.loop(0, D, step=16)
            def _(j):
                zblk[r, pl.ds(j, 16)] = jnp.zeros((16,), _f32)

        @pl.loop(0, RPT // 16)
        def _(r):
            pltpu.make_async_copy(
                zblk, acc_sh.at[pl.ds(tb + r * 16, 16)], zsem).start()

        if want_deg:
            @pl.loop(0, NPAD, step=16)
            def _(i):
                degloc[pl.ds(i, 16)] = jnp.zeros((16,), _f32)

        @pl.loop(0, RPT // 16)
        def _(r):
            pltpu.make_async_copy(
                zblk, acc_sh.at[pl.ds(tb + r * 16, 16)], zsem).wait()

        plsc.subcore_barrier()

        def load_idx(b, j):
            off = ebase + j * ECH
            pltpu.sync_copy(src_hbm.at[pl.ds(off, ECH)], sidx.at[b])
            pltpu.sync_copy(dst_hbm.at[pl.ds(off, ECH)], didx.at[b])

        def g_desc(b):
            return pltpu.make_async_copy(
                table_hbm.at[sidx.at[b]], rows.at[b], gsems[b])

        def s_desc(b):
            return pltpu.make_async_copy(
                rows.at[b], acc_sh.at[didx.at[b]], ssems[b])

        def deg_upd(b):
            if want_deg:
                for j in range(ECH // 16):
                    dv = didx[b, pl.ds(j * 16, 16)]
                    plsc.addupdate_scatter(degloc, [dv], ones16)

        def pair_body(p, first, prefetch):
            g_desc(0).wait()
            s_desc(0).start(add=True)
            deg_upd(0)
            if not first:
                s_desc(1).wait()
            load_idx(1, 2 * p + 1)
            g_desc(1).start()
            s_desc(0).wait()
            if prefetch:
                load_idx(0, 2 * p + 2)
                g_desc(0).start()
            g_desc(1).wait()
            s_desc(1).start(add=True)
            deg_upd(1)

        load_idx(0, 0)
        g_desc(0).start()
        pair_body(0, first=True, prefetch=True)

        @pl.loop(1, EMAIN // 2 - 1)
        def _(p):
            pair_body(p, first=False, prefetch=True)

        pair_body(EMAIN // 2 - 1, first=False, prefetch=False)
        s_desc(1).wait()

        toff = ebase + EMAIN * ECH
        pltpu.sync_copy(src_hbm.at[pl.ds(toff, ETAIL)], tsidx.at[0])
        pltpu.sync_copy(dst_hbm.at[pl.ds(toff, ETAIL)], tdidx.at[0])
        pltpu.sync_copy(table_hbm.at[tsidx.at[0]], trows)
        pltpu.sync_copy(trows, acc_sh.at[tdidx.at[0]], add=True)
        if want_deg:
            plsc.addupdate_scatter(degloc, [tdidx[0]], ones16[:ETAIL])

        plsc.subcore_barrier()

        pltpu.sync_copy(acc_sh.at[pl.ds(tb, RPT)],
                        acc_out.at[c, pl.ds(tb, RPT)])
        if want_deg:
            pltpu.sync_copy(degloc, deg_out.at[wid])

    return pl.kernel(body, out_type=out_types, mesh=_mesh(),
                     scratch_types=scratch_types,
                     compiler_params=_sc_params())


_agg_with_deg = _make_agg(True)
_agg_plain = _make_agg(False)


def _dense_body(relu, residual, acc0, acc1, deg, x, wlt, wrt, b, out):
    d = jnp.maximum(jnp.sum(deg[...], axis=1), 1.0)
    agg = (acc0[...] + acc1[...]) / d[:, None]
    h = (jnp.dot(agg, wlt[...], preferred_element_type=_f32)
         + jnp.dot(x[...], wrt[...], preferred_element_type=_f32)
         + b[...])
    if residual:
        h = h + x[...]
    if relu:
        h = jnp.maximum(h, 0.0)
    out[...] = h.astype(out.dtype)


def _dense(acc_p, deg_p, x, w_l, w_r, b, relu, residual, out_dtype=_f32):
    blk = 1000
    grid = (N // blk,)
    body = functools.partial(_dense_body, relu, residual)
    return pl.pallas_call(
        body,
        grid=grid,
        in_specs=[
            pl.BlockSpec((blk, D), lambda i: (i, 0)),
            pl.BlockSpec((blk, D), lambda i: (i, 0)),
            pl.BlockSpec((blk, NW), lambda i: (i, 0)),
            pl.BlockSpec((blk, D), lambda i: (i, 0)),
            pl.BlockSpec((D, D), lambda i: (0, 0)),
            pl.BlockSpec((D, D), lambda i: (0, 0)),
            pl.BlockSpec((1, D), lambda i: (0, 0)),
        ],
        out_specs=pl.BlockSpec((blk, D), lambda i: (i, 0)),
        out_shape=jax.ShapeDtypeStruct((N, D), out_dtype),
    )(acc_p[0], acc_p[1], deg_p.T, x, w_l.T, w_r.T, b.reshape(1, D))


def _decode_body(a_hbm, b_hbm, table_hbm, out_hbm, aidx, bidx, arows, brows,
                 outv, table_sh, gsem0, gsem1, isem0, isem1,
                 osem0, osem1, tsem):
    c = lax.axis_index("c")
    s = lax.axis_index("s")
    wid = s * NC + c
    nch = LBASE + jnp.where(wid < LEXTRA, 1, 0)
    gsems = (gsem0, gsem1)
    isems = (isem0, isem1)
    osems = (osem0, osem1)
    lanes = jax.lax.iota(jnp.int32, 16)

    def goff(i):
        return (wid + i * NW) * LCH

    def ia_desc(b, i):
        return pltpu.make_async_copy(
            a_hbm.at[pl.ds(goff(i), LCH)], aidx.at[b], isems[b])

    def ib_desc(b, i):
        return pltpu.make_async_copy(
            b_hbm.at[pl.ds(goff(i), LCH)], bidx.at[b], isems[b])

    def ga_desc(b):
        return pltpu.make_async_copy(
            table_sh.at[aidx.at[b]], arows.at[b], gsems[b])

    def gb_desc(b):
        return pltpu.make_async_copy(
            table_sh.at[bidx.at[b]], brows.at[b], gsems[b])

    def o_desc(b, i):
        return pltpu.make_async_copy(
            outv.at[b], out_hbm.at[pl.ds(goff(i), LCH)], osems[b])

    def compute(b):
        @pl.loop(0, LCH, step=16)
        def _(r0):
            res = jnp.zeros((16,), _f32)
            for k in range(16):
                r = r0 + k
                acc = arows[b, r, pl.ds(0, 16)] * brows[b, r, pl.ds(0, 16)]
                for j in range(1, D // 16):
                    acc = acc + (arows[b, r, pl.ds(j * 16, 16)]
                                 * brows[b, r, pl.ds(j * 16, 16)])
                res = jnp.where(lanes == k, jnp.sum(acc), res)
            outv[b, pl.ds(r0, 16)] = res

    def slot(j, b):
        @pl.when(j < nch)
        def _():
            @pl.when(j >= 2)
            def _():
                o_desc(b, j - 2).wait()

            ga_desc(b).wait()
            gb_desc(b).wait()

            @pl.when(j + 2 < nch)
            def _():
                ia_desc(b, j + 2).start()
                ib_desc(b, j + 2).start()

            b1 = b ^ 1

            @pl.when(j + 1 < nch)
            def _():
                ia_desc(b1, j + 1).wait()
                ib_desc(b1, j + 1).wait()
                ga_desc(b1).start()
                gb_desc(b1).start()

            compute(b)
            o_desc(b, j).start()

    srows = N // NS // 8 * 8 + 8
    lrows = N - 15 * srows

    @pl.when(s < 15)
    def _():
        pltpu.make_async_copy(table_hbm.at[pl.ds(s * srows, srows)],
                              table_sh.at[pl.ds(s * srows, srows)],
                              tsem).start()

    @pl.when(s == 15)
    def _():
        pltpu.make_async_copy(table_hbm.at[pl.ds(15 * srows, lrows)],
                              table_sh.at[pl.ds(15 * srows, lrows)],
                              tsem).start()

    for b in range(2):
        ia_desc(b, b).start()
        ib_desc(b, b).start()

    @pl.when(s < 15)
    def _():
        pltpu.make_async_copy(table_hbm.at[pl.ds(s * srows, srows)],
                              table_sh.at[pl.ds(s * srows, srows)],
                              tsem).wait()

    @pl.when(s == 15)
    def _():
        pltpu.make_async_copy(table_hbm.at[pl.ds(15 * srows, lrows)],
                              table_sh.at[pl.ds(15 * srows, lrows)],
                              tsem).wait()

    plsc.subcore_barrier()

    ia_desc(0, 0).wait()
    ib_desc(0, 0).wait()
    ga_desc(0).start()
    gb_desc(0).start()

    @pl.loop(0, (LBASE + 2) // 2)
    def _(t):
        j = t * 2
        slot(j, 0)
        slot(j + 1, 1)

    o_desc((LBASE - 1) % 2, LBASE - 1).wait()

    @pl.when(nch > LBASE)
    def _():
        o_desc(LBASE % 2, LBASE).wait()

    @pl.when(nch <= LBASE)
    def _():
        o_desc(LBASE % 2, LBASE - 2).wait()


_decode = pl.kernel(
    _decode_body,
    out_type=jax.ShapeDtypeStruct((EL,), _f32),
    mesh=_mesh(),
    scratch_types=[
        pltpu.VMEM((2, LCH), jnp.int32),
        pltpu.VMEM((2, LCH), jnp.int32),
        pltpu.VMEM((2, LCH, D), _f32),
        pltpu.VMEM((2, LCH, D), _f32),
        pltpu.VMEM((2, LCH), _f32),
        pltpu.VMEM_SHARED((N, D), _f32),
        pltpu.SemaphoreType.DMA,
        pltpu.SemaphoreType.DMA,
        pltpu.SemaphoreType.DMA,
        pltpu.SemaphoreType.DMA,
        pltpu.SemaphoreType.DMA,
        pltpu.SemaphoreType.DMA,
        pltpu.SemaphoreType.DMA,
    ],
    compiler_params=_sc_params(),
)


def kernel(node_features, edge_index, edge_label_index,
           W1_l, b1_l, W1_r, W2_l, b2_l, W2_r):
    src = edge_index[0]
    dst = edge_index[1]
    acc1, deg = _agg_with_deg(src, dst, node_features)
    h1 = _dense(acc1, deg, node_features, W1_l, W1_r, b1_l,
                relu=True, residual=False)
    (acc2,) = _agg_plain(src, dst, h1)
    h2 = _dense(acc2, deg, h1, W2_l, W2_r, b2_l, relu=False, residual=True)
    return _decode(edge_label_index[0], edge_label_index[1], h2)
